# baseline (speedup 1.0000x reference)
import jax
import jax.numpy as jnp
from jax import lax
from jax.experimental import pallas as pl
from jax.experimental.pallas import tpu as pltpu

N = 256
FDIM = 128
HID = 16
T = 8
BPP = 8

_BF = jnp.bfloat16


def _mm(a, b):
    return lax.dot_general(a, b, (((1,), (0,)), ((), ())),
                           preferred_element_type=jnp.float32)


def _mm_t(a, b):
    return lax.dot_general(a, b, (((0,), (0,)), ((), ())),
                           preferred_element_type=jnp.float32)


def _gru_kernel(y_ref, w0_ref, w1_ref, w2_ref, bzr_ref,
                whh_ref, bhh_ref,
                wred_ref, bred_ref, wm0_ref, bm0_ref, wm1_ref, bm1_ref,
                out_ref):
    row = lax.broadcasted_iota(jnp.int32, (N, N), 0)
    col = lax.broadcasted_iota(jnp.int32, (N, N), 1)
    offdiag = (row != col).astype(jnp.float32)

    w0 = w0_ref[...]
    w1 = w1_ref[...]
    w2 = w2_ref[...]
    bzr = bzr_ref[0]
    whh = whh_ref[...]
    bhh = bhh_ref[0]

    Hs = [jnp.zeros((N, HID), dtype=jnp.float32) for _ in range(BPP)]
    rng = range(BPP)
    for t in range(T):
        A = [y_ref[i, t, :, :N] * offdiag for i in rng]
        deg = [jnp.sum(A[i], axis=1, keepdims=True) for i in rng]
        dinv = [jnp.where(deg[i] > 0,
                          1.0 / jnp.sqrt(jnp.maximum(deg[i], 1e-12)),
                          0.0) for i in rng]
        ndrow = [jnp.transpose(-dinv[i]) for i in rng]
        Lb = [((A[i] * dinv[i]) * ndrow[i]).astype(_BF) for i in rng]

        V0 = [jnp.concatenate([y_ref[i, t, :, N:], Hs[i]], axis=1)
              for i in rng]
        V0b = [V0[i].astype(_BF) for i in rng]
        V1 = [_mm_t(Lb[i], V0b[i]) for i in rng]
        V1b = [V1[i].astype(_BF) for i in rng]
        V2 = [2.0 * _mm_t(Lb[i], V1b[i]) - V0[i] for i in rng]
        V2b = [V2[i].astype(_BF) for i in rng]
        P = [_mm(V0b[i], w0) + _mm(V1b[i], w1) + _mm(V2b[i], w2) + bzr
             for i in rng]

        Z = [jax.nn.sigmoid(P[i][:, :HID]) for i in rng]
        R = [jax.nn.sigmoid(P[i][:, HID:2 * HID]) for i in rng]

        HR = [Hs[i] * R[i] for i in rng]
        HRb = [HR[i].astype(_BF) for i in rng]
        g1 = [_mm_t(Lb[i], HRb[i]) for i in rng]
        g1b = [g1[i].astype(_BF) for i in rng]
        g2 = [(2.0 * _mm_t(Lb[i], g1b[i]) - HR[i]).astype(_BF) for i in rng]
        hcat = [jnp.concatenate([HRb[i], g1b[i], g2[i]], axis=1)
                for i in rng]
        hpre = [P[i][:, 2 * HID:] + _mm(hcat[i], whh) + bhh for i in rng]
        Htil = [jnp.tanh(hpre[i]) for i in rng]
        Hs = [Z[i] * Hs[i] + (1.0 - Z[i]) * Htil[i] for i in rng]

    for i in range(BPP):
        h = jax.nn.relu(_mm(Hs[i].astype(_BF), wred_ref[...]) + bred_ref[0])
        o = _mm_t(h.astype(_BF), wm0_ref[...]) + bm0_ref[...]
        o = _mm(o.astype(_BF), wm1_ref[...]) + bm1_ref[...]
        out_ref[i] = o


@jax.jit
def kernel(y, Wxz, bxz, Whz, bhz, Wxr, bxr, Whr, bhr, Wxh, bxh, Whh, bhh,
           Wred, bred, Wm0, bm0, Wm1, bm1):
    B = y.shape[0]
    zh = jnp.zeros((HID, HID), jnp.float32)

    def stack(wx_list, wh_list):
        top = jnp.concatenate(wx_list, axis=1)
        bot = jnp.concatenate(wh_list, axis=1)
        return jnp.concatenate([top, bot], axis=0).astype(_BF)

    w0 = stack([Wxz[0], Wxr[0], Wxh[0]], [Whz[0], Whr[0], zh])
    w1 = stack([Wxz[1], Wxr[1], Wxh[1]], [Whz[1], Whr[1], zh])
    w2 = stack([Wxz[2], Wxr[2], Wxh[2]], [Whz[2], Whr[2], zh])
    bzr = jnp.concatenate([bxz + bhz, bxr + bhr, bxh])[None, :]

    whh = jnp.concatenate([Whh[0], Whh[1], Whh[2]], axis=0).astype(_BF)
    bhh2 = bhh[None, :]
    bred2 = bred[None, :]
    bm02 = bm0[None, :]
    bm12 = bm1[None, :]

    full = lambda shape: pl.BlockSpec(shape, lambda b: (0,) * len(shape))
    out = pl.pallas_call(
        _gru_kernel,
        grid=(B // BPP,),
        in_specs=[
            pl.BlockSpec((BPP, T, N, N + FDIM), lambda b: (b, 0, 0, 0)),
            full((N // 2 + HID, 3 * HID)),
            full((N // 2 + HID, 3 * HID)),
            full((N // 2 + HID, 3 * HID)),
            full((1, 3 * HID)),
            full((3 * HID, HID)),
            full((1, HID)),
            full((HID, 1)),
            full((1, 1)),
            full((N, 32)),
            full((1, 32)),
            full((32, HID)),
            full((1, HID)),
        ],
        out_specs=pl.BlockSpec((BPP, 1, HID), lambda b: (b, 0, 0)),
        out_shape=jax.ShapeDtypeStruct((B, 1, HID), jnp.float32),
        compiler_params=pltpu.CompilerParams(
            dimension_semantics=("parallel",)),
    )(y, w0, w1, w2, bzr, whh, bhh2,
      Wred.astype(_BF), bred2, Wm0.astype(_BF), bm02, Wm1.astype(_BF), bm12)
    return out.reshape(B, HID)


# speedup vs baseline: 1.4101x; 1.4101x over previous
import jax
import jax.numpy as jnp
from jax import lax
from jax.experimental import pallas as pl
from jax.experimental.pallas import tpu as pltpu

N = 256
FDIM = 128
HID = 16
T = 8
BPP = 8

_BF = jnp.bfloat16


def _mm(a, b):
    return lax.dot_general(a, b, (((1,), (0,)), ((), ())),
                           preferred_element_type=jnp.float32)


def _mm_t(a, b):
    return lax.dot_general(a, b, (((0,), (0,)), ((), ())),
                           preferred_element_type=jnp.float32)


def _gru_kernel(y_ref, w0_ref, w1_ref, w2_ref, bzr_ref,
                whh_ref, bhh_ref,
                wred_ref, bred_ref, wm0_ref, bm0_ref, wm1_ref, bm1_ref,
                out_ref, h_ref):
    row = lax.broadcasted_iota(jnp.int32, (N, N), 0)
    col = lax.broadcasted_iota(jnp.int32, (N, N), 1)
    offdiag = (row != col).astype(jnp.float32)

    w0 = w0_ref[...]
    w1 = w1_ref[...]
    w2 = w2_ref[...]
    bzr = bzr_ref[0]
    whh = whh_ref[...]
    bhh = bhh_ref[0]

    t = pl.program_id(0)
    rng = range(BPP)

    zero16 = jnp.zeros((N, HID), dtype=jnp.float32)
    Hs = [jnp.where(t == 0, zero16, h_ref[i]) for i in rng]

    A = [y_ref[i, 0, :, :N] * offdiag for i in rng]
    deg = [jnp.sum(A[i], axis=1, keepdims=True) for i in rng]
    dinv = [jnp.where(deg[i] > 0,
                      1.0 / jnp.sqrt(jnp.maximum(deg[i], 1e-12)),
                      0.0) for i in rng]
    ndrow = [jnp.transpose(-dinv[i]) for i in rng]
    Lb = [((A[i] * dinv[i]) * ndrow[i]).astype(_BF) for i in rng]

    V0 = [jnp.concatenate([y_ref[i, 0, :, N:], Hs[i]], axis=1)
          for i in rng]
    V0b = [V0[i].astype(_BF) for i in rng]
    V1 = [_mm_t(Lb[i], V0b[i]) for i in rng]
    V1b = [V1[i].astype(_BF) for i in rng]
    V2 = [2.0 * _mm_t(Lb[i], V1b[i]) - V0[i] for i in rng]
    V2b = [V2[i].astype(_BF) for i in rng]
    P = [_mm(V0b[i], w0) + _mm(V1b[i], w1) + _mm(V2b[i], w2) + bzr
         for i in rng]

    Z = [jax.nn.sigmoid(P[i][:, :HID]) for i in rng]
    R = [jax.nn.sigmoid(P[i][:, HID:2 * HID]) for i in rng]

    HR = [Hs[i] * R[i] for i in rng]
    HRb = [HR[i].astype(_BF) for i in rng]
    g1 = [_mm_t(Lb[i], HRb[i]) for i in rng]
    g1b = [g1[i].astype(_BF) for i in rng]
    g2 = [(2.0 * _mm_t(Lb[i], g1b[i]) - HR[i]).astype(_BF) for i in rng]
    hcat = [jnp.concatenate([HRb[i], g1b[i], g2[i]], axis=1)
            for i in rng]
    hpre = [P[i][:, 2 * HID:] + _mm(hcat[i], whh) + bhh for i in rng]
    Htil = [jnp.tanh(hpre[i]) for i in rng]
    Hs = [Z[i] * Hs[i] + (1.0 - Z[i]) * Htil[i] for i in rng]
    for i in rng:
        h_ref[i] = Hs[i]

    @pl.when(t == T - 1)
    def _readout():
        for i in range(BPP):
            h = jax.nn.relu(_mm(Hs[i].astype(_BF), wred_ref[...])
                            + bred_ref[0])
            o = _mm_t(h.astype(_BF), wm0_ref[...]) + bm0_ref[...]
            o = _mm(o.astype(_BF), wm1_ref[...]) + bm1_ref[...]
            out_ref[i] = o


@jax.jit
def kernel(y, Wxz, bxz, Whz, bhz, Wxr, bxr, Whr, bhr, Wxh, bxh, Whh, bhh,
           Wred, bred, Wm0, bm0, Wm1, bm1):
    B = y.shape[0]
    zh = jnp.zeros((HID, HID), jnp.float32)

    def stack(wx_list, wh_list):
        top = jnp.concatenate(wx_list, axis=1)
        bot = jnp.concatenate(wh_list, axis=1)
        return jnp.concatenate([top, bot], axis=0).astype(_BF)

    w0 = stack([Wxz[0], Wxr[0], Wxh[0]], [Whz[0], Whr[0], zh])
    w1 = stack([Wxz[1], Wxr[1], Wxh[1]], [Whz[1], Whr[1], zh])
    w2 = stack([Wxz[2], Wxr[2], Wxh[2]], [Whz[2], Whr[2], zh])
    bzr = jnp.concatenate([bxz + bhz, bxr + bhr, bxh])[None, :]

    whh = jnp.concatenate([Whh[0], Whh[1], Whh[2]], axis=0).astype(_BF)
    bhh2 = bhh[None, :]
    bred2 = bred[None, :]
    bm02 = bm0[None, :]
    bm12 = bm1[None, :]

    full = lambda shape: pl.BlockSpec(shape, lambda b: (0,) * len(shape))
    out = pl.pallas_call(
        _gru_kernel,
        grid=(T,),
        in_specs=[
            pl.BlockSpec((BPP, 1, N, N + FDIM), lambda t: (0, t, 0, 0)),
            full((N // 2 + HID, 3 * HID)),
            full((N // 2 + HID, 3 * HID)),
            full((N // 2 + HID, 3 * HID)),
            full((1, 3 * HID)),
            full((3 * HID, HID)),
            full((1, HID)),
            full((HID, 1)),
            full((1, 1)),
            full((N, 32)),
            full((1, 32)),
            full((32, HID)),
            full((1, HID)),
        ],
        out_specs=pl.BlockSpec((BPP, 1, HID), lambda t: (0, 0, 0)),
        out_shape=jax.ShapeDtypeStruct((B, 1, HID), jnp.float32),
        scratch_shapes=[pltpu.VMEM((BPP, N, HID), jnp.float32)],
        compiler_params=pltpu.CompilerParams(
            dimension_semantics=("arbitrary",)),
    )(y, w0, w1, w2, bzr, whh, bhh2,
      Wred.astype(_BF), bred2, Wm0.astype(_BF), bm02, Wm1.astype(_BF), bm12)
    return out.reshape(B, HID)
